# baseline (speedup 1.0000x reference)
import jax
import jax.numpy as jnp
from jax.experimental import pallas as pl
from jax.experimental.pallas import tpu as pltpu

EPS = 1e-5


def _pick_tb(B):
    for tb in (16, 8, 4, 2, 1):
        if B % tb == 0 and B // tb >= 2:
            return tb
    return B


def _make_body(C, H, P, N, TB):
    PN = P * N
    O = 2 * C + 1
    inv_cnt = 1.0 / (C * PN)
    NEG = -1e30
    f32, bf16 = jnp.float32, jnp.bfloat16

    def body(x_ref, vecs_ref, wqkv_ref, wout_ref, wfc1_ref, wfc2_ref,
             segk_ref, segq_ref, o_ref):
        x = x_ref[...].reshape(TB, C, PN)

        V = vecs_ref[...]
        bqkv = V[0:O, 0:1]
        w1qkv = V[0:O, 1:2]
        bout = V[0:C, 2:3]
        bfc1 = V[0:H, 3:4]
        w1fc1 = V[0:H, 4:5]
        bfc2 = V[0:C, 5:6]

        seg_id = jax.lax.broadcasted_iota(jnp.int32, (1, 1, PN), 2) // N

        def bdot(a, b_):
            return jax.lax.dot_general(a, b_, (((2,), (1,)), ((0,), (0,))),
                                       preferred_element_type=f32)

        def wdot(w_ref, act_bf):
            w = w_ref[...]
            return bdot(jnp.broadcast_to(w[None], (TB,) + w.shape), act_bf)

        def stats(t):
            col = jnp.sum(t, axis=1, keepdims=True)
            col2 = jnp.sum(t * t, axis=1, keepdims=True)
            mu = jnp.sum(col, axis=2, keepdims=True) * inv_cnt
            ex2 = jnp.sum(col2, axis=2, keepdims=True) * inv_cnt
            rstd = jax.lax.rsqrt(ex2 - mu * mu + EPS)
            return mu, rstd

        mu1, a1 = stats(x)
        am1 = a1 * mu1
        qkv_raw = wdot(wqkv_ref, x.astype(bf16))
        k_raw = qkv_raw[:, 0:C]
        v_raw = qkv_raw[:, C:2 * C]
        q_raw = qkv_raw[:, 2 * C:2 * C + 1]

        q = a1 * q_raw + (bqkv[2 * C:2 * C + 1] - am1 * w1qkv[2 * C:2 * C + 1])

        shift = jnp.zeros_like(q)
        for p in range(P):
            m = seg_id == p
            pmax = jnp.max(jnp.where(m, q, NEG), axis=2, keepdims=True)
            shift = jnp.where(m, pmax, shift)
        e = jnp.exp(q - shift)
        dens = []
        for p in range(P):
            m = seg_id == p
            dens.append(jnp.sum(jnp.where(m, e, 0.0), axis=2, keepdims=True))
        den = jnp.concatenate(dens, axis=2)

        ke = (k_raw * e).astype(bf16)
        sums = bdot(ke, jnp.broadcast_to(segk_ref[...][None], (TB, PN, P)))
        ctx = a1 * (sums * pl.reciprocal(den, approx=True)) \
            + (bqkv[0:C] - am1 * w1qkv[0:C])
        ctx_full = bdot(ctx.astype(bf16),
                        jnp.broadcast_to(segq_ref[...][None], (TB, P, PN)))

        v = jnp.maximum(a1 * v_raw + (bqkv[C:2 * C] - am1 * w1qkv[C:2 * C]), 0.0)
        attn = wdot(wout_ref, (v * ctx_full).astype(bf16)) + bout
        x2 = x + attn

        mu2, a2 = stats(x2)
        h = a2 * wdot(wfc1_ref, x2.astype(bf16)) + (bfc1 - a2 * mu2 * w1fc1)
        h = h * jax.nn.sigmoid(h)
        mlp = wdot(wfc2_ref, h.astype(bf16)) + bfc2
        o_ref[...] = (x2 + mlp).astype(o_ref.dtype).reshape(TB, C, P, N)

    return body


def kernel(x, g1, b1, wqkv, bqkv, wout, bout, g2, b2, wfc1, bfc1, wfc2, bfc2):
    B, C, P, N = x.shape
    PN = P * N
    H = wfc1.shape[0]
    O = 2 * C + 1
    f32 = jnp.float32

    w_r = jnp.concatenate([wqkv[1:1 + C], wqkv[1 + C:], wqkv[0:1]], axis=0)
    b_r = jnp.concatenate([bqkv[1:1 + C], bqkv[1 + C:], bqkv[0:1]], axis=0)

    wqkv_f = (w_r.astype(f32) * g1[None, :].astype(f32)).astype(jnp.bfloat16)
    wfc1_f = (wfc1.astype(f32) * g2[None, :].astype(f32)).astype(jnp.bfloat16)
    bqkv_f = b_r.astype(f32) + w_r.astype(f32) @ b1.astype(f32)
    bfc1_f = bfc1.astype(f32) + wfc1.astype(f32) @ b2.astype(f32)
    w1qkv = jnp.sum(wqkv_f.astype(f32), axis=1)
    w1fc1 = jnp.sum(wfc1_f.astype(f32), axis=1)

    Rmax = max(O, H, C)

    def padcol(a):
        a = a.astype(f32).reshape(-1)
        return jnp.pad(a, (0, Rmax - a.shape[0]))

    vecs = jnp.stack([padcol(bqkv_f), padcol(w1qkv), padcol(bout),
                      padcol(bfc1_f), padcol(w1fc1), padcol(bfc2),
                      padcol(bfc2), padcol(bfc2)], axis=1)

    wout_b = wout.astype(jnp.bfloat16)
    wfc2_b = wfc2.astype(jnp.bfloat16)

    TB = _pick_tb(B)
    G = B // TB

    li = jnp.arange(PN, dtype=jnp.int32)
    segk = (li[:, None] // N == jnp.arange(P, dtype=jnp.int32)[None, :])
    segk = segk.astype(jnp.bfloat16)
    segq = segk.T

    full = lambda a: pl.BlockSpec(a.shape, lambda i: (0,) * a.ndim)

    out = pl.pallas_call(
        _make_body(C, H, P, N, TB),
        out_shape=jax.ShapeDtypeStruct((B, C, P, N), x.dtype),
        grid=(G,),
        in_specs=[pl.BlockSpec((TB, C, P, N), lambda i: (i, 0, 0, 0)),
                  full(vecs), full(wqkv_f), full(wout_b),
                  full(wfc1_f), full(wfc2_b), full(segk), full(segq)],
        out_specs=pl.BlockSpec((TB, C, P, N), lambda i: (i, 0, 0, 0)),
        compiler_params=pltpu.CompilerParams(
            dimension_semantics=("parallel",),
            vmem_limit_bytes=48 * 1024 * 1024),
    )(x, vecs, wqkv_f, wout_b, wfc1_f, wfc2_b, segk, segq)

    return out


# speedup vs baseline: 3.1999x; 3.1999x over previous
import jax
import jax.numpy as jnp
from jax.experimental import pallas as pl
from jax.experimental.pallas import tpu as pltpu

EPS = 1e-5


def _pick_tb(B):
    for tb in (16, 8, 4, 2, 1):
        if B % tb == 0 and B // tb >= 2:
            return tb
    return B


def _make_body(C, H, P, N, TB):
    PN = P * N
    L = TB * PN
    S = TB * P
    O = 2 * C + 1
    inv_cnt = 1.0 / (C * PN)
    f32, bf16 = jnp.float32, jnp.bfloat16

    def body(x_ref, vecs_ref, wqkv_ref, wout_ref, wfc1_ref, wfc2_ref,
             segs_ref, segb_ref, sel_ref, o_ref):
        x3 = x_ref[...]

        V = vecs_ref[...]
        bqkv, w1qkv = V[0:1], V[1:2]
        bout = V[2:3, 0:C]
        bfc1, w1fc1 = V[3:4, 0:H], V[4:5, 0:H]
        bfc2 = V[5:6, 0:C]

        def dot(a, b_):
            return jax.lax.dot_general(a, b_, (((1,), (0,)), ((), ())),
                                       preferred_element_type=f32)

        def stats(t3):
            col = jnp.sum(t3, axis=1, keepdims=True)
            col2 = jnp.sum(t3 * t3, axis=1, keepdims=True)
            mu = jnp.sum(col, axis=2, keepdims=True) * inv_cnt
            ex2 = jnp.sum(col2, axis=2, keepdims=True) * inv_cnt
            rstd = jax.lax.rsqrt(ex2 - mu * mu + EPS)
            return mu, rstd

        def rows(t):
            return jnp.broadcast_to(t, (TB, PN, 1)).reshape(L, 1)

        mu1, a1 = stats(x3)
        a1r, am1r = rows(a1), rows(a1 * mu1)
        qkv = dot(x3.astype(bf16).reshape(L, C), wqkv_ref[...])
        k_raw = qkv[:, 0:C]
        v_raw = qkv[:, C:2 * C]
        q_raw = qkv[:, 2 * C:2 * C + 1]

        q = a1r * q_raw + (bqkv[:, 2 * C:] - am1r * w1qkv[:, 2 * C:])
        qm = dot(segs_ref[...], q.astype(bf16)) * (1.0 / N)
        shift = dot(segb_ref[...], qm.astype(bf16))
        e = jnp.exp(q - shift)
        e_bf = e.astype(bf16)
        den = dot(segs_ref[...], e_bf)

        ke = (k_raw * e).astype(bf16)
        sums = dot(segs_ref[...], ke)
        a_seg = dot(sel_ref[...], jnp.concatenate(
            [a1.reshape(TB, 1), (a1 * mu1).reshape(TB, 1)], axis=1))
        ctx = a_seg[:, 0:1] * (sums * pl.reciprocal(den, approx=True)) \
            + (bqkv[:, 0:C] - a_seg[:, 1:2] * w1qkv[:, 0:C])
        ctx_full = dot(segb_ref[...], ctx.astype(bf16))

        v = jnp.maximum(a1r * v_raw + (bqkv[:, C:2 * C] - am1r * w1qkv[:, C:2 * C]), 0.0)
        attn = dot((v * ctx_full).astype(bf16), wout_ref[...]) + bout
        x2 = x3 + attn.reshape(TB, PN, C)

        mu2, a2 = stats(x2)
        h_raw = dot(x2.astype(bf16).reshape(L, C), wfc1_ref[...])
        h = rows(a2) * h_raw + (bfc1 - rows(a2 * mu2) * w1fc1)
        hh = 0.5 * h
        h = hh + hh * jnp.tanh(hh)
        mlp = dot(h.astype(bf16), wfc2_ref[...]) + bfc2
        o_ref[...] = (x2 + mlp.reshape(TB, PN, C)).astype(o_ref.dtype)

    return body


def kernel(x, g1, b1, wqkv, bqkv, wout, bout, g2, b2, wfc1, bfc1, wfc2, bfc2):
    B, C, P, N = x.shape
    PN = P * N
    H = wfc1.shape[0]
    O = 2 * C + 1
    f32 = jnp.float32

    w_r = jnp.concatenate([wqkv[1:1 + C], wqkv[1 + C:], wqkv[0:1]], axis=0)
    b_r = jnp.concatenate([bqkv[1:1 + C], bqkv[1 + C:], bqkv[0:1]], axis=0)

    wqkv_f = (w_r.astype(f32) * g1[None, :].astype(f32)).astype(jnp.bfloat16)
    wfc1_f = (wfc1.astype(f32) * g2[None, :].astype(f32)).astype(jnp.bfloat16)
    bqkv_f = b_r.astype(f32) + w_r.astype(f32) @ b1.astype(f32)
    bfc1_f = bfc1.astype(f32) + wfc1.astype(f32) @ b2.astype(f32)
    w1qkv = jnp.sum(wqkv_f.astype(f32), axis=1)
    w1fc1 = jnp.sum(wfc1_f.astype(f32), axis=1)

    Omax = max(O, H)

    def padrow(a):
        a = a.astype(f32).reshape(-1)
        return jnp.pad(a, (0, Omax - a.shape[0]))

    vecs = jnp.stack([padrow(bqkv_f), padrow(w1qkv), padrow(bout),
                      padrow(bfc1_f), padrow(w1fc1), padrow(bfc2),
                      padrow(bfc2), padrow(bfc2)], axis=0)

    TB = _pick_tb(B)
    G = B // TB
    L = TB * PN
    S = TB * P

    x_cl = jnp.transpose(x, (0, 2, 3, 1)).reshape(B, PN, C)

    seg_eq = (jnp.arange(L, dtype=jnp.int32)[:, None] // N
              == jnp.arange(S, dtype=jnp.int32)[None, :])
    segs = seg_eq.T.astype(jnp.bfloat16)
    segb = seg_eq.astype(jnp.bfloat16)
    sel = (jnp.arange(S, dtype=jnp.int32)[:, None] // P
           == jnp.arange(TB, dtype=jnp.int32)[None, :]).astype(f32)

    bf16 = jnp.bfloat16
    ins = [x_cl, vecs, wqkv_f.T, wout.astype(bf16).T, wfc1_f.T,
           wfc2.astype(bf16).T, segs, segb, sel]

    full = lambda a: pl.BlockSpec(a.shape, lambda i: (0,) * a.ndim)

    out = pl.pallas_call(
        _make_body(C, H, P, N, TB),
        out_shape=jax.ShapeDtypeStruct((B, PN, C), x.dtype),
        grid=(G,),
        in_specs=[pl.BlockSpec((TB, PN, C), lambda i: (i, 0, 0))]
                 + [full(a) for a in ins[1:]],
        out_specs=pl.BlockSpec((TB, PN, C), lambda i: (i, 0, 0)),
        compiler_params=pltpu.CompilerParams(
            dimension_semantics=("parallel",),
            vmem_limit_bytes=48 * 1024 * 1024),
    )(*ins)

    return jnp.transpose(out.reshape(B, P, N, C), (0, 3, 1, 2))


# speedup vs baseline: 3.4148x; 1.0671x over previous
import jax
import jax.numpy as jnp
from jax.experimental import pallas as pl
from jax.experimental.pallas import tpu as pltpu

EPS = 1e-5


def _pick_tb(B):
    for tb in (16, 8, 4, 2, 1):
        if B % tb == 0 and B // tb >= 2:
            return tb
    return B


def _make_body(C, H, P, N, TB):
    PN = P * N
    L = TB * PN
    S = TB * P
    O = 2 * C + 1
    inv_cnt = 1.0 / (C * PN)
    f32, bf16 = jnp.float32, jnp.bfloat16

    def body(x_ref, bqkv_ref, w1qkv_ref, bout_ref, bfc1_ref, w1fc1_ref,
             bfc2_ref, wqkv_ref, wout_ref, wfc1_ref, wfc2_ref,
             segs_ref, segb_ref, sel_ref, o_ref):
        x3 = x_ref[...]
        bqkv, w1qkv = bqkv_ref[...], w1qkv_ref[...]

        def dot(a, b_):
            return jax.lax.dot_general(a, b_, (((1,), (0,)), ((), ())),
                                       preferred_element_type=f32)

        def stats(t3):
            col = jnp.sum(t3, axis=1, keepdims=True)
            col2 = jnp.sum(t3 * t3, axis=1, keepdims=True)
            mu = jnp.sum(col, axis=2, keepdims=True) * inv_cnt
            ex2 = jnp.sum(col2, axis=2, keepdims=True) * inv_cnt
            rstd = jax.lax.rsqrt(ex2 - mu * mu + EPS)
            return mu, rstd

        def rows(t):
            return jnp.broadcast_to(t, (TB, PN, 1)).reshape(L, 1)

        mu1, a1 = stats(x3)
        a1r, am1r = rows(a1), rows(a1 * mu1)
        qkv = dot(x3.astype(bf16).reshape(L, C), wqkv_ref[...])
        k_raw = qkv[:, 0:C]
        v_raw = qkv[:, C:2 * C]
        q_raw = qkv[:, 2 * C:2 * C + 1]

        q = a1r * q_raw + (bqkv[:, 2 * C:] - am1r * w1qkv[:, 2 * C:])
        q4 = q.reshape(TB, P, N, 1)
        qm = jnp.sum(q4, axis=2, keepdims=True) * (1.0 / N)
        e4 = jnp.exp(q4 - qm)
        e = e4.reshape(L, 1)
        den = dot(segs_ref[...], e.astype(bf16))

        ke = (k_raw * e).astype(bf16)
        sums = dot(segs_ref[...], ke)
        a_seg = dot(sel_ref[...], jnp.concatenate(
            [a1.reshape(TB, 1), (a1 * mu1).reshape(TB, 1)], axis=1))
        ctx = a_seg[:, 0:1] * (sums * pl.reciprocal(den, approx=True)) \
            + (bqkv[:, 0:C] - a_seg[:, 1:2] * w1qkv[:, 0:C])
        ctx_full = dot(segb_ref[...], ctx.astype(bf16))

        v = jnp.maximum(a1r * v_raw + (bqkv[:, C:2 * C] - am1r * w1qkv[:, C:2 * C]), 0.0)
        attn = dot((v * ctx_full).astype(bf16), wout_ref[...]) + bout_ref[...]
        x2 = x3 + attn.reshape(TB, PN, C)

        mu2, a2 = stats(x2)
        h_raw = dot(x2.astype(bf16).reshape(L, C), wfc1_ref[...])
        h = rows(a2) * h_raw + (bfc1_ref[...] - rows(a2 * mu2) * w1fc1_ref[...])
        hh = 0.5 * h
        h = hh + hh * jnp.tanh(hh)
        mlp = dot(h.astype(bf16), wfc2_ref[...]) + bfc2_ref[...]
        o_ref[...] = (x2 + mlp.reshape(TB, PN, C)).astype(o_ref.dtype)

    return body


def kernel(x, g1, b1, wqkv, bqkv, wout, bout, g2, b2, wfc1, bfc1, wfc2, bfc2):
    B, C, P, N = x.shape
    PN = P * N
    H = wfc1.shape[0]
    O = 2 * C + 1
    f32 = jnp.float32

    w_r = jnp.concatenate([wqkv[1:1 + C], wqkv[1 + C:], wqkv[0:1]], axis=0)
    b_r = jnp.concatenate([bqkv[1:1 + C], bqkv[1 + C:], bqkv[0:1]], axis=0)

    wqkv_f = (w_r.astype(f32) * g1[None, :].astype(f32)).astype(jnp.bfloat16)
    wfc1_f = (wfc1.astype(f32) * g2[None, :].astype(f32)).astype(jnp.bfloat16)
    bqkv_f = b_r.astype(f32) + w_r.astype(f32) @ b1.astype(f32)
    bfc1_f = bfc1.astype(f32) + wfc1.astype(f32) @ b2.astype(f32)
    w1qkv = jnp.sum(wqkv_f.astype(f32), axis=1)
    w1fc1 = jnp.sum(wfc1_f.astype(f32), axis=1)

    row = lambda a: a.astype(f32).reshape(1, -1)

    TB = _pick_tb(B)
    G = B // TB
    L = TB * PN
    S = TB * P

    x_cl = jnp.transpose(x, (0, 2, 3, 1)).reshape(B, PN, C)

    seg_eq = (jnp.arange(L, dtype=jnp.int32)[:, None] // N
              == jnp.arange(S, dtype=jnp.int32)[None, :])
    segs = seg_eq.T.astype(jnp.bfloat16)
    segb = seg_eq.astype(jnp.bfloat16)
    sel = (jnp.arange(S, dtype=jnp.int32)[:, None] // P
           == jnp.arange(TB, dtype=jnp.int32)[None, :]).astype(f32)

    bf16 = jnp.bfloat16
    ins = [x_cl, row(bqkv_f), row(w1qkv), row(bout), row(bfc1_f), row(w1fc1),
           row(bfc2), wqkv_f.T, wout.astype(bf16).T, wfc1_f.T,
           wfc2.astype(bf16).T, segs, segb, sel]

    full = lambda a: pl.BlockSpec(a.shape, lambda i: (0,) * a.ndim)

    out = pl.pallas_call(
        _make_body(C, H, P, N, TB),
        out_shape=jax.ShapeDtypeStruct((B, PN, C), x.dtype),
        grid=(G,),
        in_specs=[pl.BlockSpec((TB, PN, C), lambda i: (i, 0, 0))]
                 + [full(a) for a in ins[1:]],
        out_specs=pl.BlockSpec((TB, PN, C), lambda i: (i, 0, 0)),
        compiler_params=pltpu.CompilerParams(
            dimension_semantics=("parallel",),
            vmem_limit_bytes=48 * 1024 * 1024),
    )(*ins)

    return jnp.transpose(out.reshape(B, P, N, C), (0, 3, 1, 2))
